# baseline (speedup 1.0000x reference)
import functools

import jax
import jax.numpy as jnp
from jax import lax
from jax.experimental import pallas as pl
from jax.experimental.pallas import tpu as pltpu
from jax.experimental.pallas import tpu_sc as plsc

_NC = 2
_NS = 16
_NW = _NC * _NS
_L = 16


def kernel(inputs, pos_table):
    B, S, D = inputs.shape
    flat_in = inputs.reshape(B, S * D)
    flat_pos = pos_table.reshape(S * D)

    CH = 16
    rows_per_w = S // _NW
    n_chunks = rows_per_w // CH
    chunk_elems = CH * D

    mesh = plsc.VectorSubcoreMesh(core_axis_name="c", subcore_axis_name="s")

    @functools.partial(
        pl.kernel,
        mesh=mesh,
        out_type=jax.ShapeDtypeStruct((B, S * D), jnp.float32),
        scratch_types=[
            pltpu.VMEM((chunk_elems,), jnp.float32),
            pltpu.VMEM((chunk_elems,), jnp.float32),
        ],
    )
    def sc_add(in_hbm, pos_hbm, out_hbm, pos_v, data_v):
        wid = lax.axis_index("c") * _NS + lax.axis_index("s")
        base = wid * rows_per_w * D

        def chunk_body(c, carry):
            off = base + c * chunk_elems
            pltpu.sync_copy(pos_hbm.at[pl.ds(off, chunk_elems)], pos_v)
            for b in range(B):
                pltpu.sync_copy(in_hbm.at[b, pl.ds(off, chunk_elems)], data_v)

                def vec_body(i, carry2):
                    j = i * (_L * 8)
                    for u in range(8):
                        jj = j + u * _L
                        data_v[pl.ds(jj, _L)] = (
                            data_v[pl.ds(jj, _L)] + pos_v[pl.ds(jj, _L)]
                        )
                    return carry2

                lax.fori_loop(0, chunk_elems // (_L * 8), vec_body, 0)
                pltpu.sync_copy(data_v, out_hbm.at[b, pl.ds(off, chunk_elems)])
            return carry

        lax.fori_loop(0, n_chunks, chunk_body, 0)

    out = sc_add(flat_in, flat_pos)
    return out.reshape(B, S, D)
